# baseline (speedup 1.0000x reference)
import functools

import jax
import jax.numpy as jnp
from jax import lax
from jax.experimental import pallas as pl
from jax.experimental.pallas import tpu as pltpu
from jax.experimental.pallas import tpu_sc as plsc

D_IN_ = 768
D_SAE_ = 24576
K_ = 64
N_TOK_ = 8192


SC_NW = 32
SC_ROWCHUNK = 32


def _xstats_sc(x):
    n_tok, d_in = x.shape
    rows_per_w = n_tok // SC_NW
    mesh = plsc.VectorSubcoreMesh(core_axis_name="c", subcore_axis_name="s")

    @functools.partial(
        pl.kernel,
        mesh=mesh,
        out_type=[
            jax.ShapeDtypeStruct((SC_NW, d_in), jnp.float32),
            jax.ShapeDtypeStruct((SC_NW, d_in), jnp.float32),
        ],
        scratch_types=[
            pltpu.VMEM((SC_ROWCHUNK, d_in), jnp.float32),
            pltpu.VMEM((d_in,), jnp.float32),
            pltpu.VMEM((d_in,), jnp.float32),
            pltpu.SemaphoreType.DMA,
        ],
    )
    def k(x_hbm, xs_hbm, xsq_hbm, buf, s1v, s2v, sem):
        wid = lax.axis_index("s") * 2 + lax.axis_index("c")
        base = wid * rows_per_w
        zeros = jnp.zeros((16,), jnp.float32)

        @pl.loop(0, d_in, step=16)
        def _(j):
            s1v[pl.ds(j, 16)] = zeros
            s2v[pl.ds(j, 16)] = zeros

        @pl.loop(0, rows_per_w, step=SC_ROWCHUNK)
        def _(r0):
            pltpu.async_copy(
                x_hbm.at[pl.ds(base + r0, SC_ROWCHUNK)], buf, sem).wait()

            @pl.loop(0, SC_ROWCHUNK)
            def _(r):
                @pl.loop(0, d_in, step=16)
                def _(j):
                    v = buf[r, pl.ds(j, 16)]
                    s1v[pl.ds(j, 16)] = s1v[pl.ds(j, 16)] + v
                    s2v[pl.ds(j, 16)] = s2v[pl.ds(j, 16)] + v * v

        pltpu.async_copy(s1v, xs_hbm.at[wid], sem).wait()
        pltpu.async_copy(s2v, xsq_hbm.at[wid], sem).wait()

    return k(x)


TB_ENC = 512
FB_ENC = 4096


def _enc_body(x_ref, w_ref, be_ref, bd_ref, h_ref):
    xb = x_ref[...] - bd_ref[...][None, :]
    acc = jnp.dot(xb, w_ref[...], preferred_element_type=jnp.float32)
    h_ref[...] = jnp.maximum(acc + be_ref[...][None, :], 0.0)


def _encode(x, W_enc, b_enc, b_dec):
    n_tok, d_in = x.shape
    d_sae = W_enc.shape[1]
    grid = (d_sae // FB_ENC, n_tok // TB_ENC)
    return pl.pallas_call(
        _enc_body,
        grid=grid,
        in_specs=[
            pl.BlockSpec((TB_ENC, d_in), lambda j, t: (t, 0)),
            pl.BlockSpec((d_in, FB_ENC), lambda j, t: (0, j)),
            pl.BlockSpec((FB_ENC,), lambda j, t: (j,)),
            pl.BlockSpec((d_in,), lambda j, t: (0,)),
        ],
        out_specs=pl.BlockSpec((TB_ENC, FB_ENC), lambda j, t: (t, j)),
        out_shape=jax.ShapeDtypeStruct((n_tok, d_sae), jnp.float32),
    )(x, W_enc, b_enc, b_dec)


TB_SEL = 64


def _sel_body(h_ref, fa_ref):
    h = h_ref[...]
    tb = h.shape[0]
    bits = jax.lax.bitcast_convert_type(h, jnp.int32)

    CW = 768

    def count_ge(data16, mid16):
        acc = (data16[:, :CW] > mid16).astype(jnp.int16)
        for c in range(1, data16.shape[1] // CW):
            acc = acc + (data16[:, c * CW:(c + 1) * CW] > mid16).astype(jnp.int16)
        return jnp.sum(acc.astype(jnp.int32), axis=1, keepdims=True)

    half = tb // 2

    def bisect2(data16, lo0, hi0, n_pass):
        d_a, d_b = data16[:half], data16[half:]

        def step(_, c):
            lo_a, hi_a, lo_b, hi_b = c
            mid_a = (lo_a + hi_a) >> 1
            mid_b = (lo_b + hi_b) >> 1
            ge_a = count_ge(d_a, mid_a.astype(jnp.int16)) >= K_
            ge_b = count_ge(d_b, mid_b.astype(jnp.int16)) >= K_
            return (jnp.where(ge_a, mid_a, lo_a), jnp.where(ge_a, hi_a, mid_a),
                    jnp.where(ge_b, mid_b, lo_b), jnp.where(ge_b, hi_b, mid_b))

        lo_a = jnp.full((half, 1), lo0, jnp.int32)
        hi_a = jnp.full((half, 1), hi0, jnp.int32)
        out = jax.lax.fori_loop(0, n_pass, step, (lo_a, hi_a, lo_a, hi_a))
        return jnp.concatenate([out[0], out[2]], axis=0)

    hi16 = (bits >> 16).astype(jnp.int16)
    lo1 = bisect2(hi16, -1, 0x7F80, 15)
    h_star16 = (lo1 + 1).astype(jnp.int16)
    base = (lo1 + 1) << 16

    y16 = bits.astype(jnp.int16) ^ jnp.int16(-32768)
    gt = hi16 > h_star16
    eq = hi16 == h_star16
    z = jnp.where(gt, jnp.int16(32767), jnp.where(eq, y16, jnp.int16(-32768)))

    lo2 = bisect2(z, -32769, 32767, 16)
    have_low = lo2 >= -32768

    low = lo2 + 32768
    thr_bits = jnp.where(have_low, base + low, base - 1)
    thr_bits = jnp.maximum(thr_bits, 0)
    thr = jax.lax.bitcast_convert_type(thr_bits, jnp.float32)
    fa_ref[...] = jnp.where(h > thr, h, 0.0)


def _select(hidden):
    n_tok, d_sae = hidden.shape
    return pl.pallas_call(
        _sel_body,
        grid=(n_tok // TB_SEL,),
        in_specs=[pl.BlockSpec((TB_SEL, d_sae), lambda t: (t, 0))],
        out_specs=pl.BlockSpec((TB_SEL, d_sae), lambda t: (t, 0)),
        out_shape=jax.ShapeDtypeStruct((n_tok, d_sae), jnp.float32),
    )(hidden)


TB_DEC = 1024
KB_DEC = 2048


def _dec_body(fa_ref, w_ref, x_ref, bd_ref, xsp_ref, xsqp_ref, o_ref,
              loss_ref, fvu_ref, *, n_k, n_t, n_tok):
    t = pl.program_id(0)
    kc = pl.program_id(1)
    acc = jnp.dot(fa_ref[...].astype(jnp.bfloat16), w_ref[...],
                  preferred_element_type=jnp.float32)

    @pl.when(kc == 0)
    def _():
        o_ref[...] = acc

    @pl.when(kc != 0)
    def _():
        o_ref[...] += acc

    @pl.when(kc == n_k - 1)
    def _():
        out = o_ref[...] + bd_ref[...][None, :]
        o_ref[...] = out
        d2 = jnp.sum((out - x_ref[...]) ** 2, axis=0, keepdims=True)

        @pl.when(t == 0)
        def _():
            loss_ref[...] = d2

        @pl.when(t != 0)
        def _():
            loss_ref[...] += d2

        @pl.when(t == n_t - 1)
        def _():
            xs = jnp.sum(xsp_ref[...], axis=0, keepdims=True)
            xsq = jnp.sum(xsqp_ref[...], axis=0, keepdims=True)
            tv = xsq - (xs / n_tok) * xs
            fvu_ref[...] = jnp.mean(loss_ref[...] / tv).reshape(1, 1)


def _decode(feature_acts, W_dec, x, b_dec, xs_p, xsq_p):
    n_tok, d_sae = feature_acts.shape
    d_in = W_dec.shape[1]
    n_t = n_tok // TB_DEC
    n_k = d_sae // KB_DEC
    body = functools.partial(_dec_body, n_k=n_k, n_t=n_t, n_tok=n_tok)
    return pl.pallas_call(
        body,
        grid=(n_t, n_k),
        in_specs=[
            pl.BlockSpec((TB_DEC, KB_DEC), lambda t, kc: (t, kc)),
            pl.BlockSpec((KB_DEC, d_in), lambda t, kc: (kc, 0)),
            pl.BlockSpec((TB_DEC, d_in), lambda t, kc: (t, 0)),
            pl.BlockSpec((d_in,), lambda t, kc: (0,)),
            pl.BlockSpec((SC_NW, d_in), lambda t, kc: (0, 0)),
            pl.BlockSpec((SC_NW, d_in), lambda t, kc: (0, 0)),
        ],
        out_specs=[
            pl.BlockSpec((TB_DEC, d_in), lambda t, kc: (t, 0)),
            pl.BlockSpec((1, d_in), lambda t, kc: (0, 0)),
            pl.BlockSpec((1, 1), lambda t, kc: (0, 0)),
        ],
        out_shape=[
            jax.ShapeDtypeStruct((n_tok, d_in), jnp.float32),
            jax.ShapeDtypeStruct((1, d_in), jnp.float32),
            jax.ShapeDtypeStruct((1, 1), jnp.float32),
        ],
    )(feature_acts, W_dec, x, b_dec, xs_p, xsq_p)


def kernel(x, W_enc, W_dec, b_enc, b_dec):
    xs_p, xsq_p = _xstats_sc(x)
    hidden = _encode(x, W_enc, b_enc, b_dec)
    feature_acts = _select(hidden)
    sae_out, _, fvu = _decode(feature_acts, W_dec.astype(jnp.bfloat16),
                              x, b_dec, xs_p, xsq_p)
    return (sae_out, feature_acts, fvu[0, 0])


# speedup vs baseline: 1.0024x; 1.0024x over previous
import functools

import jax
import jax.numpy as jnp
from jax import lax
from jax.experimental import pallas as pl
from jax.experimental.pallas import tpu as pltpu
from jax.experimental.pallas import tpu_sc as plsc

D_IN_ = 768
D_SAE_ = 24576
K_ = 64
N_TOK_ = 8192


SC_NW = 32
SC_ROWCHUNK = 32


def _xstats_sc(x):
    n_tok, d_in = x.shape
    rows_per_w = n_tok // SC_NW
    mesh = plsc.VectorSubcoreMesh(core_axis_name="c", subcore_axis_name="s")

    @functools.partial(
        pl.kernel,
        mesh=mesh,
        out_type=[
            jax.ShapeDtypeStruct((SC_NW, d_in), jnp.float32),
            jax.ShapeDtypeStruct((SC_NW, d_in), jnp.float32),
        ],
        scratch_types=[
            pltpu.VMEM((SC_ROWCHUNK, d_in), jnp.float32),
            pltpu.VMEM((d_in,), jnp.float32),
            pltpu.VMEM((d_in,), jnp.float32),
            pltpu.SemaphoreType.DMA,
        ],
    )
    def k(x_hbm, xs_hbm, xsq_hbm, buf, s1v, s2v, sem):
        wid = lax.axis_index("s") * 2 + lax.axis_index("c")
        base = wid * rows_per_w
        zeros = jnp.zeros((16,), jnp.float32)

        @pl.loop(0, d_in, step=16)
        def _(j):
            s1v[pl.ds(j, 16)] = zeros
            s2v[pl.ds(j, 16)] = zeros

        @pl.loop(0, rows_per_w, step=SC_ROWCHUNK)
        def _(r0):
            pltpu.async_copy(
                x_hbm.at[pl.ds(base + r0, SC_ROWCHUNK)], buf, sem).wait()

            @pl.loop(0, SC_ROWCHUNK)
            def _(r):
                @pl.loop(0, d_in, step=16)
                def _(j):
                    v = buf[r, pl.ds(j, 16)]
                    s1v[pl.ds(j, 16)] = s1v[pl.ds(j, 16)] + v
                    s2v[pl.ds(j, 16)] = s2v[pl.ds(j, 16)] + v * v

        pltpu.async_copy(s1v, xs_hbm.at[wid], sem).wait()
        pltpu.async_copy(s2v, xsq_hbm.at[wid], sem).wait()

    return k(x)


TB_ENC = 512
FB_ENC = 4096


def _enc_body(x_ref, w_ref, be_ref, bd_ref, h_ref):
    xb = x_ref[...] - bd_ref[...][None, :]
    acc = jnp.dot(xb, w_ref[...], preferred_element_type=jnp.float32)
    h_ref[...] = jnp.maximum(acc + be_ref[...][None, :], 0.0)


def _encode(x, W_enc, b_enc, b_dec):
    n_tok, d_in = x.shape
    d_sae = W_enc.shape[1]
    grid = (d_sae // FB_ENC, n_tok // TB_ENC)
    return pl.pallas_call(
        _enc_body,
        grid=grid,
        in_specs=[
            pl.BlockSpec((TB_ENC, d_in), lambda j, t: (t, 0)),
            pl.BlockSpec((d_in, FB_ENC), lambda j, t: (0, j)),
            pl.BlockSpec((FB_ENC,), lambda j, t: (j,)),
            pl.BlockSpec((d_in,), lambda j, t: (0,)),
        ],
        out_specs=pl.BlockSpec((TB_ENC, FB_ENC), lambda j, t: (t, j)),
        out_shape=jax.ShapeDtypeStruct((n_tok, d_sae), jnp.float32),
    )(x, W_enc, b_enc, b_dec)


TB_SEL = 64


def _sel_body(h_ref, fa_ref):
    h = h_ref[...]
    tb = h.shape[0]
    bits = jax.lax.bitcast_convert_type(h, jnp.int32)

    CW = 768

    def count_ge(data16, mid16):
        acc = (data16[:, :CW] > mid16).astype(jnp.int16)
        for c in range(1, data16.shape[1] // CW):
            acc = acc + (data16[:, c * CW:(c + 1) * CW] > mid16).astype(jnp.int16)
        return jnp.sum(acc.astype(jnp.int32), axis=1, keepdims=True)

    hi16 = (bits >> 16).astype(jnp.int16)

    def s1(_, carry):
        lo, hi = carry
        mid = (lo + hi) >> 1
        ge = count_ge(hi16, mid.astype(jnp.int16)) >= K_
        return (jnp.where(ge, mid, lo), jnp.where(ge, hi, mid))

    lo1 = jnp.full((tb, 1), -1, jnp.int32)
    hi1 = jnp.full((tb, 1), 0x7F80, jnp.int32)
    lo1, _ = jax.lax.fori_loop(0, 15, s1, (lo1, hi1))
    h_star16 = (lo1 + 1).astype(jnp.int16)
    base = (lo1 + 1) << 16

    y16 = bits.astype(jnp.int16) ^ jnp.int16(-32768)
    gt = hi16 > h_star16
    eq = hi16 == h_star16
    z = jnp.where(gt, jnp.int16(32767), jnp.where(eq, y16, jnp.int16(-32768)))

    def s2(_, carry):
        lo, hi = carry
        mid = (lo + hi) >> 1
        ge = count_ge(z, mid.astype(jnp.int16)) >= K_
        return (jnp.where(ge, mid, lo), jnp.where(ge, hi, mid))

    lo2 = jnp.full((tb, 1), -32769, jnp.int32)
    hi2 = jnp.full((tb, 1), 32767, jnp.int32)
    lo2, _ = jax.lax.fori_loop(0, 16, s2, (lo2, hi2))
    have_low = lo2 >= -32768

    low = lo2 + 32768
    thr_bits = jnp.where(have_low, base + low, base - 1)
    thr_bits = jnp.maximum(thr_bits, 0)
    thr = jax.lax.bitcast_convert_type(thr_bits, jnp.float32)
    fa_ref[...] = jnp.where(h > thr, h, 0.0)


def _select(hidden):
    n_tok, d_sae = hidden.shape
    return pl.pallas_call(
        _sel_body,
        grid=(n_tok // TB_SEL,),
        in_specs=[pl.BlockSpec((TB_SEL, d_sae), lambda t: (t, 0))],
        out_specs=pl.BlockSpec((TB_SEL, d_sae), lambda t: (t, 0)),
        out_shape=jax.ShapeDtypeStruct((n_tok, d_sae), jnp.float32),
    )(hidden)


TB_DEC = 1024
KB_DEC = 2048


def _dec_body(fa_ref, w_ref, x_ref, bd_ref, xsp_ref, xsqp_ref, o_ref,
              loss_ref, fvu_ref, *, n_k, n_t, n_tok):
    t = pl.program_id(0)
    kc = pl.program_id(1)
    acc = jnp.dot(fa_ref[...].astype(jnp.bfloat16), w_ref[...],
                  preferred_element_type=jnp.float32)

    @pl.when(kc == 0)
    def _():
        o_ref[...] = acc

    @pl.when(kc != 0)
    def _():
        o_ref[...] += acc

    @pl.when(kc == n_k - 1)
    def _():
        out = o_ref[...] + bd_ref[...][None, :]
        o_ref[...] = out
        d2 = jnp.sum((out - x_ref[...]) ** 2, axis=0, keepdims=True)

        @pl.when(t == 0)
        def _():
            loss_ref[...] = d2

        @pl.when(t != 0)
        def _():
            loss_ref[...] += d2

        @pl.when(t == n_t - 1)
        def _():
            xs = jnp.sum(xsp_ref[...], axis=0, keepdims=True)
            xsq = jnp.sum(xsqp_ref[...], axis=0, keepdims=True)
            tv = xsq - (xs / n_tok) * xs
            fvu_ref[...] = jnp.mean(loss_ref[...] / tv).reshape(1, 1)


def _decode(feature_acts, W_dec, x, b_dec, xs_p, xsq_p):
    n_tok, d_sae = feature_acts.shape
    d_in = W_dec.shape[1]
    n_t = n_tok // TB_DEC
    n_k = d_sae // KB_DEC
    body = functools.partial(_dec_body, n_k=n_k, n_t=n_t, n_tok=n_tok)
    return pl.pallas_call(
        body,
        grid=(n_t, n_k),
        in_specs=[
            pl.BlockSpec((TB_DEC, KB_DEC), lambda t, kc: (t, kc)),
            pl.BlockSpec((KB_DEC, d_in), lambda t, kc: (kc, 0)),
            pl.BlockSpec((TB_DEC, d_in), lambda t, kc: (t, 0)),
            pl.BlockSpec((d_in,), lambda t, kc: (0,)),
            pl.BlockSpec((SC_NW, d_in), lambda t, kc: (0, 0)),
            pl.BlockSpec((SC_NW, d_in), lambda t, kc: (0, 0)),
        ],
        out_specs=[
            pl.BlockSpec((TB_DEC, d_in), lambda t, kc: (t, 0)),
            pl.BlockSpec((1, d_in), lambda t, kc: (0, 0)),
            pl.BlockSpec((1, 1), lambda t, kc: (0, 0)),
        ],
        out_shape=[
            jax.ShapeDtypeStruct((n_tok, d_in), jnp.float32),
            jax.ShapeDtypeStruct((1, d_in), jnp.float32),
            jax.ShapeDtypeStruct((1, 1), jnp.float32),
        ],
    )(feature_acts, W_dec, x, b_dec, xs_p, xsq_p)


def kernel(x, W_enc, W_dec, b_enc, b_dec):
    xs_p, xsq_p = _xstats_sc(x)
    hidden = _encode(x, W_enc, b_enc, b_dec)
    feature_acts = _select(hidden)
    sae_out, _, fvu = _decode(feature_acts, W_dec.astype(jnp.bfloat16),
                              x, b_dec, xs_p, xsq_p)
    return (sae_out, feature_acts, fvu[0, 0])
